# Initial kernel scaffold; baseline (speedup 1.0000x reference)
import jax
import jax.numpy as jnp
from jax.experimental import pallas as pl


def kernel(x, edge_index, edge_attr, u, batch, W1, b1, W2, b2, W3, b3, W4, b4, W5, b5):
    raise NotImplementedError("write your pallas kernel here")


# speedup vs baseline: 6.5156x; 6.5156x over previous
import functools

import jax
import jax.numpy as jnp
from jax import lax
from jax.experimental import pallas as pl
from jax.experimental.pallas import tpu as pltpu
from jax.experimental.pallas import tpu_sc as plsc

N = 100000
E = 3200000
H = 16
NC = 2
NS = 16
NW = NC * NS
EPW = E // NW
CE = 2000
NCH = EPW // CE
NPZ = N // NS
NUB = 4000
NUBW = N // NUB

_mesh = plsc.VectorSubcoreMesh(core_axis_name="c", subcore_axis_name="s")


def _p_body(x_ref, w_ref, b_ref, o_ref):
    o_ref[...] = (
        jnp.dot(x_ref[...], w_ref[...], preferred_element_type=jnp.float32, precision="highest")
        + b_ref[...]
    )


def _node_pre(x, w1a, b1):
    bn = 4000
    return pl.pallas_call(
        _p_body,
        grid=(N // bn,),
        in_specs=[
            pl.BlockSpec((bn, 2), lambda i: (i, 0)),
            pl.BlockSpec((2, H), lambda i: (0, 0)),
            pl.BlockSpec((1, H), lambda i: (0, 0)),
        ],
        out_specs=pl.BlockSpec((bn, H), lambda i: (i, 0)),
        out_shape=jax.ShapeDtypeStruct((N, H), jnp.float32),
    )(x, w1a, b1.reshape(1, H))


GR = 80
NG = CE // GR


@functools.partial(
    pl.kernel,
    mesh=_mesh,
    out_type=jax.ShapeDtypeStruct((E, H), jnp.float32),
    scratch_types=[
        pltpu.VMEM((NG, GR), jnp.int32),
        pltpu.VMEM((CE, H), jnp.float32),
        pltpu.SemaphoreType.DMA,
    ],
    compiler_params=pltpu.CompilerParams(use_tc_tiling_on_sc=False),
)
def _sc_gather(p_hbm, src2_hbm, out_hbm, idx_v, rows_v, sem):
    wid = lax.axis_index("s") * NC + lax.axis_index("c")
    base = wid * EPW

    def body(i, carry):
        off = base + i * CE
        pltpu.sync_copy(src2_hbm.at[pl.ds(off // GR, NG)], idx_v)

        def fire(r, c):
            pltpu.async_copy(
                p_hbm.at[idx_v.at[r]], rows_v.at[pl.ds(r * GR, GR)], sem
            )
            return c

        def drain(r, c):
            pltpu.make_async_copy(
                p_hbm.at[idx_v.at[r]], rows_v.at[pl.ds(r * GR, GR)], sem
            ).wait()
            return c

        lax.fori_loop(0, NG, fire, 0)
        lax.fori_loop(0, NG, drain, 0)
        pltpu.sync_copy(rows_v, out_hbm.at[pl.ds(off, CE)])
        return carry

    lax.fori_loop(0, NCH, body, 0)


def _mlp_body(g_ref, ea_ref, s1_ref, w2_ref, b2_ref, w3_ref, b3_ref, o_ref):
    h1 = g_ref[...] + jnp.dot(
        ea_ref[...], s1_ref[...], preferred_element_type=jnp.float32, precision="highest"
    )
    h1 = jnp.maximum(h1, 0.0)
    h2 = (
        jnp.dot(h1, w2_ref[...], preferred_element_type=jnp.float32, precision="highest")
        + b2_ref[...]
    )
    h2 = jnp.maximum(h2, 0.0)
    o_ref[...] = (
        jnp.dot(h2, w3_ref[...], preferred_element_type=jnp.float32, precision="highest")
        + b3_ref[...]
    )


def _edge_mlp(gp, ea8, s1, w2d, b2t, w3d, b3t):
    rows = E // 8
    br = 1000
    return pl.pallas_call(
        _mlp_body,
        grid=(rows // br,),
        in_specs=[
            pl.BlockSpec((br, 128), lambda i: (i, 0)),
            pl.BlockSpec((br, 8), lambda i: (i, 0)),
            pl.BlockSpec((8, 128), lambda i: (0, 0)),
            pl.BlockSpec((128, 128), lambda i: (0, 0)),
            pl.BlockSpec((1, 128), lambda i: (0, 0)),
            pl.BlockSpec((128, 128), lambda i: (0, 0)),
            pl.BlockSpec((1, 128), lambda i: (0, 0)),
        ],
        out_specs=pl.BlockSpec((br, 128), lambda i: (i, 0)),
        out_shape=jax.ShapeDtypeStruct((rows, 128), jnp.float32),
    )(gp, ea8, s1, w2d, b2t, w3d, b3t)


N2 = N // NC
PAD = 2048
M = N2 + PAD
MZ = M // NS
EPT = E // NS
NCH2 = EPT // CE
NPC = N2 // NS


@functools.partial(
    pl.kernel,
    mesh=_mesh,
    out_type=(
        jax.ShapeDtypeStruct((N, H), jnp.float32),
        jax.ShapeDtypeStruct((N,), jnp.float32),
        jax.ShapeDtypeStruct((N,), jnp.float32),
    ),
    scratch_types=[
        pltpu.VMEM((NG, GR), jnp.int32),
        pltpu.VMEM((CE, H), jnp.float32),
        pltpu.VMEM((GR,), jnp.float32),
        pltpu.VMEM((NUB // GR, GR), jnp.int32),
        pltpu.VMEM((NUB,), jnp.float32),
        pltpu.VMEM_SHARED((M, H), jnp.float32),
        pltpu.VMEM_SHARED((M,), jnp.float32),
        pltpu.SemaphoreType.DMA,
    ],
    compiler_params=pltpu.CompilerParams(use_tc_tiling_on_sc=False),
)
def _sc_scatter(
    h3_hbm, dst2_hbm, batch2_hbm, u_hbm, z16_hbm, z1_hbm, one_hbm,
    sums_out, cnt_out, ub_out,
    idx_v, val_v, ones_v, bat_v, ub_v, acc_sh, cnt_sh, sem,
):
    cid = lax.axis_index("c")
    sid = lax.axis_index("s")
    wid = sid * NC + cid
    lo = cid * N2

    pltpu.sync_copy(z16_hbm.at[pl.ds(0, MZ)], acc_sh.at[pl.ds(sid * MZ, MZ)])

    @pl.when(sid == 0)
    def _():
        pltpu.sync_copy(z1_hbm, cnt_sh)

    pltpu.sync_copy(one_hbm, ones_v)
    plsc.subcore_barrier()

    base = sid * EPT

    def body(i, carry):
        off = base + i * CE
        pltpu.sync_copy(dst2_hbm.at[pl.ds(off // GR, NG)], idx_v)
        pltpu.sync_copy(h3_hbm.at[pl.ds(off, CE)], val_v)

        def remap(k, c):
            r = k // (GR // 16)
            j = k % (GR // 16)
            v = idx_v[r, pl.ds(j * 16, 16)]
            local = v - lo
            keep = (local >= 0) & (local < N2)
            pad = N2 + (v & (PAD - 1))
            idx_v[r, pl.ds(j * 16, 16)] = jnp.where(keep, local, pad)
            return c

        lax.fori_loop(0, NG * (GR // 16), remap, 0)

        def scat(r, c):
            pltpu.sync_copy(
                val_v.at[pl.ds(r * GR, GR)], acc_sh.at[idx_v.at[r]], add=True
            )
            pltpu.sync_copy(ones_v, cnt_sh.at[idx_v.at[r]], add=True)
            return c

        lax.fori_loop(0, NG, scat, 0)
        return carry

    lax.fori_loop(0, NCH2, body, 0)
    plsc.subcore_barrier()

    rb = sid * NPC
    pltpu.sync_copy(acc_sh.at[pl.ds(rb, NPC)], sums_out.at[pl.ds(lo + rb, NPC)])

    @pl.when(sid == 0)
    def _():
        pltpu.sync_copy(cnt_sh.at[pl.ds(0, N2)], cnt_out.at[pl.ds(lo, N2)])

    @pl.when(wid < NUBW)
    def _():
        nb = wid * NUB
        nr = NUB // GR
        pltpu.sync_copy(batch2_hbm.at[pl.ds(nb // GR, nr)], bat_v)

        def fire(r, c):
            pltpu.async_copy(
                u_hbm.at[bat_v.at[r]], ub_v.at[pl.ds(r * GR, GR)], sem
            )
            return c

        def drain(r, c):
            pltpu.make_async_copy(
                u_hbm.at[bat_v.at[r]], ub_v.at[pl.ds(r * GR, GR)], sem
            ).wait()
            return c

        lax.fori_loop(0, nr, fire, 0)
        lax.fori_loop(0, nr, drain, 0)
        pltpu.sync_copy(ub_v, ub_out.at[pl.ds(nb, NUB)])


def _node_body(
    x_ref, s_ref, c_ref, ub_ref,
    w4a_ref, w4b_ref, w4c_ref, b4_ref, w5_ref, b5_ref, o_ref,
):
    cnt = jnp.maximum(c_ref[...], 1.0)
    mean = s_ref[...] / cnt
    h4 = (
        jnp.dot(x_ref[...], w4a_ref[...], preferred_element_type=jnp.float32, precision="highest")
        + jnp.dot(mean, w4b_ref[...], preferred_element_type=jnp.float32, precision="highest")
        + ub_ref[...] * w4c_ref[...]
        + b4_ref[...]
    )
    h4 = jnp.maximum(h4, 0.0)
    o_ref[...] = (
        jnp.dot(h4, w5_ref[...], preferred_element_type=jnp.float32, precision="highest")
        + b5_ref[...]
    )


def _node_mlp(x, s, c, ub, w4a, w4b, w4c, b4, w5, b5):
    bn = 4000
    return pl.pallas_call(
        _node_body,
        grid=(N // bn,),
        in_specs=[
            pl.BlockSpec((bn, 2), lambda i: (i, 0)),
            pl.BlockSpec((bn, H), lambda i: (i, 0)),
            pl.BlockSpec((bn, 1), lambda i: (i, 0)),
            pl.BlockSpec((bn, 1), lambda i: (i, 0)),
            pl.BlockSpec((2, 32), lambda i: (0, 0)),
            pl.BlockSpec((H, 32), lambda i: (0, 0)),
            pl.BlockSpec((1, 32), lambda i: (0, 0)),
            pl.BlockSpec((1, 32), lambda i: (0, 0)),
            pl.BlockSpec((32, 2), lambda i: (0, 0)),
            pl.BlockSpec((1, 2), lambda i: (0, 0)),
        ],
        out_specs=pl.BlockSpec((bn, 2), lambda i: (i, 0)),
        out_shape=jax.ShapeDtypeStruct((N, 2), jnp.float32),
    )(x, s, c, ub, w4a, w4b, w4c, b4, w5, b5)


def kernel(x, edge_index, edge_attr, u, batch, W1, b1, W2, b2, W3, b3, W4, b4, W5, b5):
    src = edge_index[0]
    dst = edge_index[1]

    p = _node_pre(x, W1[:2], b1)

    g = _sc_gather(p, src.reshape(E // GR, GR))

    eye8 = jnp.eye(8, dtype=jnp.float32)
    s1 = jnp.kron(eye8, W1[2:3])
    w2d = jnp.kron(eye8, W2)
    w3d = jnp.kron(eye8, W3)
    b2t = jnp.tile(b2, 8).reshape(1, 128)
    b3t = jnp.tile(b3, 8).reshape(1, 128)
    gp = g.reshape(E // 8, 128)
    ea8 = edge_attr.reshape(E // 8, 8)
    h3 = _edge_mlp(gp, ea8, s1, w2d, b2t, w3d, b3t).reshape(E, H)

    z16 = jnp.zeros((MZ + 8, H), jnp.float32)
    z1 = jnp.zeros((M,), jnp.float32)
    one = jnp.ones((GR,), jnp.float32)
    sums, cnts, ub = _sc_scatter(
        h3, dst.reshape(E // GR, GR), batch.reshape(N // GR, GR), u,
        z16, z1, one,
    )

    pad_c = lambda a: jnp.pad(a, ((0, 0), (0, 32 - a.shape[1])))
    w4a = pad_c(W4[:2])
    w4b = pad_c(W4[2:2 + H])
    w4c = pad_c(W4[2 + H:2 + H + 1])
    b4p = pad_c(b4.reshape(1, -1))
    w5p = jnp.pad(W5, ((0, 32 - W5.shape[0]), (0, 0)))
    return _node_mlp(
        x, sums, cnts.reshape(N, 1), ub.reshape(N, 1),
        w4a, w4b, w4c, b4p, w5p, b5.reshape(1, 2),
    )


# speedup vs baseline: 9.7201x; 1.4918x over previous
import functools

import jax
import jax.numpy as jnp
from jax import lax
from jax.experimental import pallas as pl
from jax.experimental.pallas import tpu as pltpu
from jax.experimental.pallas import tpu_sc as plsc

N = 100000
E = 3200000
H = 16
NC = 2
NS = 16
NW = NC * NS
EPW = E // NW
CE = 2000
NCH = EPW // CE
NPZ = N // NS
NUB = 4000
NUBW = N // NUB

_mesh = plsc.VectorSubcoreMesh(core_axis_name="c", subcore_axis_name="s")


def _p_body(x_ref, w_ref, b_ref, o_ref):
    o_ref[...] = (
        jnp.dot(x_ref[...], w_ref[...], preferred_element_type=jnp.float32, precision="highest")
        + b_ref[...]
    )


def _node_pre(x, w1a, b1):
    bn = 4000
    return pl.pallas_call(
        _p_body,
        grid=(N // bn,),
        in_specs=[
            pl.BlockSpec((bn, 2), lambda i: (i, 0)),
            pl.BlockSpec((2, H), lambda i: (0, 0)),
            pl.BlockSpec((1, H), lambda i: (0, 0)),
        ],
        out_specs=pl.BlockSpec((bn, H), lambda i: (i, 0)),
        out_shape=jax.ShapeDtypeStruct((N, H), jnp.float32),
    )(x, w1a, b1.reshape(1, H))


GR = 80
NG = CE // GR


@functools.partial(
    pl.kernel,
    mesh=_mesh,
    out_type=jax.ShapeDtypeStruct((E, H), jnp.float32),
    scratch_types=[
        pltpu.VMEM((NG, GR), jnp.int32),
        pltpu.VMEM((CE, H), jnp.float32),
        pltpu.SemaphoreType.DMA,
    ],
    compiler_params=pltpu.CompilerParams(use_tc_tiling_on_sc=False),
)
def _sc_gather(p_hbm, src2_hbm, out_hbm, idx_v, rows_v, sem):
    wid = lax.axis_index("s") * NC + lax.axis_index("c")
    base = wid * EPW

    def body(i, carry):
        off = base + i * CE
        pltpu.sync_copy(src2_hbm.at[pl.ds(off // GR, NG)], idx_v)

        def fire(r, c):
            pltpu.async_copy(
                p_hbm.at[idx_v.at[r]], rows_v.at[pl.ds(r * GR, GR)], sem
            )
            return c

        def drain(r, c):
            pltpu.make_async_copy(
                p_hbm.at[idx_v.at[r]], rows_v.at[pl.ds(r * GR, GR)], sem
            ).wait()
            return c

        lax.fori_loop(0, NG, fire, 0)
        lax.fori_loop(0, NG, drain, 0)
        pltpu.sync_copy(rows_v, out_hbm.at[pl.ds(off, CE)])
        return carry

    lax.fori_loop(0, NCH, body, 0)


def _mlp_body(g_ref, ea_ref, s1_ref, w2_ref, b2_ref, w3_ref, b3_ref, o_ref):
    h1 = g_ref[...] + jnp.dot(
        ea_ref[...], s1_ref[...], preferred_element_type=jnp.float32, precision="highest"
    )
    h1 = jnp.maximum(h1, 0.0)
    h2 = (
        jnp.dot(h1, w2_ref[...], preferred_element_type=jnp.float32, precision="highest")
        + b2_ref[...]
    )
    h2 = jnp.maximum(h2, 0.0)
    o_ref[...] = (
        jnp.dot(h2, w3_ref[...], preferred_element_type=jnp.float32, precision="highest")
        + b3_ref[...]
    )


def _edge_mlp(gp, ea8, s1, w2d, b2t, w3d, b3t):
    rows = E // 8
    br = 8000
    return pl.pallas_call(
        _mlp_body,
        grid=(rows // br,),
        in_specs=[
            pl.BlockSpec((br, 128), lambda i: (i, 0)),
            pl.BlockSpec((br, 8), lambda i: (i, 0)),
            pl.BlockSpec((8, 128), lambda i: (0, 0)),
            pl.BlockSpec((128, 128), lambda i: (0, 0)),
            pl.BlockSpec((1, 128), lambda i: (0, 0)),
            pl.BlockSpec((128, 128), lambda i: (0, 0)),
            pl.BlockSpec((1, 128), lambda i: (0, 0)),
        ],
        out_specs=pl.BlockSpec((br, 128), lambda i: (i, 0)),
        out_shape=jax.ShapeDtypeStruct((rows, 128), jnp.float32),
    )(gp, ea8, s1, w2d, b2t, w3d, b3t)


N2 = N // NC
PAD = 2048
M = N2 + PAD
MZ = M // NS
EPT = E // NS
NCH2 = EPT // CE
NPC = N2 // NS


@functools.partial(
    pl.kernel,
    mesh=_mesh,
    out_type=(
        jax.ShapeDtypeStruct((N, H), jnp.float32),
        jax.ShapeDtypeStruct((N,), jnp.float32),
        jax.ShapeDtypeStruct((N,), jnp.float32),
    ),
    scratch_types=[
        pltpu.VMEM((NG, GR), jnp.int32),
        pltpu.VMEM((CE, H), jnp.float32),
        pltpu.VMEM((GR,), jnp.float32),
        pltpu.VMEM((NUB // GR, GR), jnp.int32),
        pltpu.VMEM((NUB,), jnp.float32),
        pltpu.VMEM_SHARED((M, H), jnp.float32),
        pltpu.VMEM_SHARED((M,), jnp.float32),
        pltpu.SemaphoreType.DMA,
        pltpu.SemaphoreType.DMA,
    ],
    compiler_params=pltpu.CompilerParams(use_tc_tiling_on_sc=False),
)
def _sc_scatter(
    h3_hbm, dst2_hbm, batch2_hbm, u_hbm, z16_hbm, z1_hbm, one_hbm,
    sums_out, cnt_out, ub_out,
    idx_v, val_v, ones_v, bat_v, ub_v, acc_sh, cnt_sh, sem, sem2,
):
    cid = lax.axis_index("c")
    sid = lax.axis_index("s")
    wid = sid * NC + cid
    lo = cid * N2

    pltpu.sync_copy(z16_hbm.at[pl.ds(0, MZ)], acc_sh.at[pl.ds(sid * MZ, MZ)])

    @pl.when(sid == 0)
    def _():
        pltpu.sync_copy(z1_hbm, cnt_sh)

    pltpu.sync_copy(one_hbm, ones_v)
    plsc.subcore_barrier()

    base = sid * EPT

    def body(i, carry):
        off = base + i * CE
        pltpu.sync_copy(dst2_hbm.at[pl.ds(off // GR, NG)], idx_v)
        pltpu.sync_copy(h3_hbm.at[pl.ds(off, CE)], val_v)

        def remap(k, c):
            r = k // (GR // 16)
            j = k % (GR // 16)
            v = idx_v[r, pl.ds(j * 16, 16)]
            local = v - lo
            keep = (local >= 0) & (local < N2)
            pad = N2 + (v & (PAD - 1))
            idx_v[r, pl.ds(j * 16, 16)] = jnp.where(keep, local, pad)
            return c

        lax.fori_loop(0, NG * (GR // 16), remap, 0)

        def scat_fire(r, c):
            pltpu.async_copy(
                val_v.at[pl.ds(r * GR, GR)], acc_sh.at[idx_v.at[r]], sem,
                add=True,
            )
            pltpu.async_copy(
                ones_v, cnt_sh.at[idx_v.at[r]], sem2, add=True
            )
            return c

        def scat_drain(r, c):
            pltpu.make_async_copy(
                val_v.at[pl.ds(r * GR, GR)], acc_sh.at[idx_v.at[r]], sem
            ).wait()
            pltpu.make_async_copy(
                ones_v, cnt_sh.at[idx_v.at[r]], sem2
            ).wait()
            return c

        lax.fori_loop(0, NG, scat_fire, 0)
        lax.fori_loop(0, NG, scat_drain, 0)
        return carry

    lax.fori_loop(0, NCH2, body, 0)
    plsc.subcore_barrier()

    rb = sid * NPC
    pltpu.sync_copy(acc_sh.at[pl.ds(rb, NPC)], sums_out.at[pl.ds(lo + rb, NPC)])

    @pl.when(sid == 0)
    def _():
        pltpu.sync_copy(cnt_sh.at[pl.ds(0, N2)], cnt_out.at[pl.ds(lo, N2)])

    @pl.when(wid < NUBW)
    def _():
        nb = wid * NUB
        nr = NUB // GR
        pltpu.sync_copy(batch2_hbm.at[pl.ds(nb // GR, nr)], bat_v)

        def fire(r, c):
            pltpu.async_copy(
                u_hbm.at[bat_v.at[r]], ub_v.at[pl.ds(r * GR, GR)], sem
            )
            return c

        def drain(r, c):
            pltpu.make_async_copy(
                u_hbm.at[bat_v.at[r]], ub_v.at[pl.ds(r * GR, GR)], sem
            ).wait()
            return c

        lax.fori_loop(0, nr, fire, 0)
        lax.fori_loop(0, nr, drain, 0)
        pltpu.sync_copy(ub_v, ub_out.at[pl.ds(nb, NUB)])


def _node_body(
    x_ref, s_ref, c_ref, ub_ref,
    w4a_ref, w4b_ref, w4c_ref, b4_ref, w5_ref, b5_ref, o_ref,
):
    cnt = jnp.maximum(c_ref[...], 1.0)
    mean = s_ref[...] / cnt
    h4 = (
        jnp.dot(x_ref[...], w4a_ref[...], preferred_element_type=jnp.float32, precision="highest")
        + jnp.dot(mean, w4b_ref[...], preferred_element_type=jnp.float32, precision="highest")
        + ub_ref[...] * w4c_ref[...]
        + b4_ref[...]
    )
    h4 = jnp.maximum(h4, 0.0)
    o_ref[...] = (
        jnp.dot(h4, w5_ref[...], preferred_element_type=jnp.float32, precision="highest")
        + b5_ref[...]
    )


def _node_mlp(x, s, c, ub, w4a, w4b, w4c, b4, w5, b5):
    bn = 4000
    return pl.pallas_call(
        _node_body,
        grid=(N // bn,),
        in_specs=[
            pl.BlockSpec((bn, 2), lambda i: (i, 0)),
            pl.BlockSpec((bn, H), lambda i: (i, 0)),
            pl.BlockSpec((bn, 1), lambda i: (i, 0)),
            pl.BlockSpec((bn, 1), lambda i: (i, 0)),
            pl.BlockSpec((2, 32), lambda i: (0, 0)),
            pl.BlockSpec((H, 32), lambda i: (0, 0)),
            pl.BlockSpec((1, 32), lambda i: (0, 0)),
            pl.BlockSpec((1, 32), lambda i: (0, 0)),
            pl.BlockSpec((32, 2), lambda i: (0, 0)),
            pl.BlockSpec((1, 2), lambda i: (0, 0)),
        ],
        out_specs=pl.BlockSpec((bn, 2), lambda i: (i, 0)),
        out_shape=jax.ShapeDtypeStruct((N, 2), jnp.float32),
    )(x, s, c, ub, w4a, w4b, w4c, b4, w5, b5)


def kernel(x, edge_index, edge_attr, u, batch, W1, b1, W2, b2, W3, b3, W4, b4, W5, b5):
    src = edge_index[0]
    dst = edge_index[1]

    p = _node_pre(x, W1[:2], b1)

    g = _sc_gather(p, src.reshape(E // GR, GR))

    eye8 = jnp.eye(8, dtype=jnp.float32)
    s1 = jnp.kron(eye8, W1[2:3])
    w2d = jnp.kron(eye8, W2)
    w3d = jnp.kron(eye8, W3)
    b2t = jnp.tile(b2, 8).reshape(1, 128)
    b3t = jnp.tile(b3, 8).reshape(1, 128)
    gp = g.reshape(E // 8, 128)
    ea8 = edge_attr.reshape(E // 8, 8)
    h3 = _edge_mlp(gp, ea8, s1, w2d, b2t, w3d, b3t).reshape(E, H)

    z16 = jnp.zeros((MZ + 8, H), jnp.float32)
    z1 = jnp.zeros((M,), jnp.float32)
    one = jnp.ones((GR,), jnp.float32)
    sums, cnts, ub = _sc_scatter(
        h3, dst.reshape(E // GR, GR), batch.reshape(N // GR, GR), u,
        z16, z1, one,
    )

    pad_c = lambda a: jnp.pad(a, ((0, 0), (0, 32 - a.shape[1])))
    w4a = pad_c(W4[:2])
    w4b = pad_c(W4[2:2 + H])
    w4c = pad_c(W4[2 + H:2 + H + 1])
    b4p = pad_c(b4.reshape(1, -1))
    w5p = jnp.pad(W5, ((0, 32 - W5.shape[0]), (0, 0)))
    return _node_mlp(
        x, sums, cnts.reshape(N, 1), ub.reshape(N, 1),
        w4a, w4b, w4c, b4p, w5p, b5.reshape(1, 2),
    )
